# baseline (speedup 1.0000x reference)
import functools

import jax
import jax.numpy as jnp
from jax import lax
from jax.experimental import pallas as pl
from jax.experimental.pallas import tpu as pltpu
from jax.experimental.pallas import tpu_sc as plsc

N_NODES = 10000
N_EDGES = 320000
CH = 128
N_GRAPHS = 64
N_CLASSES = 10

NC, NS = 2, 16
NW = NC * NS
WIN = 112
NWIN = 90
DEG_GRP = 10
EPAD = NW * NWIN * WIN
NROWS = EPAD // WIN
NPAD = 10240
ROWS_PER_TILE = NPAD // NS

BLK = 1024
GRID = NPAD // BLK

f32 = jnp.float32


@functools.cache
def _build_sc_kernels():
    mesh = plsc.VectorSubcoreMesh(core_axis_name="c", subcore_axis_name="s",
                                  num_cores=NC, num_subcores=NS)

    @functools.partial(
        pl.kernel,
        out_type=jax.ShapeDtypeStruct((NC, NPAD), f32),
        mesh=mesh,
        scratch_types=[
            pltpu.VMEM((NWIN, 2, WIN), jnp.int32),
            pltpu.VMEM((WIN,), f32),
            pltpu.VMEM_SHARED((NPAD,), f32),
            pltpu.SemaphoreType.DMA,
        ],
    )
    def _sc_degree(eidx_hbm, init_hbm, out_hbm, dst_v, ones_v, deg_sh, sem):
        c = lax.axis_index("c")
        s = lax.axis_index("s")
        wid = s * NC + c
        pltpu.sync_copy(eidx_hbm.at[pl.ds(wid * NWIN, NWIN)], dst_v)
        for i in range(WIN // 16):
            ones_v[pl.ds(i * 16, 16)] = jnp.full((16,), 1.0, f32)
        base = s * ROWS_PER_TILE
        pltpu.sync_copy(init_hbm.at[c, pl.ds(base, ROWS_PER_TILE)],
                        deg_sh.at[pl.ds(base, ROWS_PER_TILE)])
        plsc.subcore_barrier()

        def scat(j):
            return pltpu.make_async_copy(ones_v, deg_sh.at[dst_v.at[j, 1]],
                                         sem)

        def body(g, carry):
            for i in range(DEG_GRP):
                scat(g * DEG_GRP + i).start(add=True)
            for i in range(DEG_GRP):
                scat(g * DEG_GRP + i).wait()
            return carry

        lax.fori_loop(0, NWIN // DEG_GRP, body, 0)
        plsc.subcore_barrier()
        pltpu.sync_copy(deg_sh.at[pl.ds(base, ROWS_PER_TILE)],
                        out_hbm.at[c, pl.ds(base, ROWS_PER_TILE)])

    @functools.partial(
        pl.kernel,
        out_type=jax.ShapeDtypeStruct((NC, NPAD, CH), f32),
        mesh=mesh,
        scratch_types=[
            pltpu.VMEM((5, 2, WIN), jnp.int32),
            pltpu.VMEM((3, WIN, CH), f32),
            pltpu.VMEM_SHARED((NPAD, CH), f32),
            pltpu.SemaphoreType.DMA,
            pltpu.SemaphoreType.DMA,
            pltpu.SemaphoreType.DMA,
        ],
    )
    def _sc_spmm(eidx_hbm, yw_hbm, out_hbm, idx_v, rows_v, acc_sh,
                 isem, gsem, ssem):
        c = lax.axis_index("c")
        s = lax.axis_index("s")
        wid = s * NC + c
        row0 = wid * NWIN
        base = s * ROWS_PER_TILE

        def idx_copy(j):
            return pltpu.make_async_copy(eidx_hbm.at[row0 + j],
                                         idx_v.at[j % 5], isem)

        def gather(j):
            return pltpu.make_async_copy(yw_hbm.at[idx_v.at[j % 5, 0]],
                                         rows_v.at[j % 3], gsem)

        def scatter(j):
            return pltpu.make_async_copy(rows_v.at[j % 3],
                                         acc_sh.at[idx_v.at[j % 5, 1]], ssem)

        idx_copy(0).start()
        idx_copy(1).start()
        pltpu.sync_copy(yw_hbm.at[pl.ds(base, ROWS_PER_TILE)],
                        acc_sh.at[pl.ds(base, ROWS_PER_TILE)])
        plsc.subcore_barrier()
        idx_copy(0).wait()
        gather(0).start()

        def body(j, carry):
            @pl.when(j >= 2)
            def _():
                scatter(j - 2).wait()

            @pl.when(j + 2 < NWIN)
            def _():
                idx_copy(j + 2).start()

            @pl.when(j + 1 < NWIN)
            def _():
                idx_copy(j + 1).wait()
                gather(j + 1).start()

            gather(j).wait()
            scatter(j).start(add=True)
            return carry

        lax.fori_loop(0, NWIN, body, 0)

        @pl.when(NWIN >= 2)
        def _():
            scatter(NWIN - 2).wait()

        scatter(NWIN - 1).wait()
        plsc.subcore_barrier()
        pltpu.sync_copy(acc_sh.at[pl.ds(base, ROWS_PER_TILE)],
                        out_hbm.at[c, pl.ds(base, ROWS_PER_TILE)])

    return _sc_degree, _sc_spmm


def _tca_body(x_ref, w1_ref, xw_ref):
    xw_ref[...] = jnp.dot(x_ref[...], w1_ref[...], preferred_element_type=f32)


_tca = pl.pallas_call(
    _tca_body,
    grid=(GRID,),
    in_specs=[
        pl.BlockSpec((BLK, CH), lambda i: (i, 0)),
        pl.BlockSpec((CH, CH), lambda i: (0, 0)),
    ],
    out_specs=pl.BlockSpec((BLK, CH), lambda i: (i, 0)),
    out_shape=jax.ShapeDtypeStruct((NPAD, CH), f32),
)


def _tcb_body(deg_ref, xw_ref, yw_ref, dinv_ref):
    deg = deg_ref[0] + deg_ref[1]
    dinv = jnp.where(deg > 0, lax.rsqrt(deg), 0.0)
    dinv_ref[...] = dinv
    yw_ref[...] = dinv * xw_ref[...]


_tcb = pl.pallas_call(
    _tcb_body,
    grid=(GRID,),
    in_specs=[
        pl.BlockSpec((NC, BLK, 1), lambda i: (0, i, 0)),
        pl.BlockSpec((BLK, CH), lambda i: (i, 0)),
    ],
    out_specs=[
        pl.BlockSpec((BLK, CH), lambda i: (i, 0)),
        pl.BlockSpec((BLK, 1), lambda i: (i, 0)),
    ],
    out_shape=[
        jax.ShapeDtypeStruct((NPAD, CH), f32),
        jax.ShapeDtypeStruct((NPAD, 1), f32),
    ],
)


def _tc2_body(acc_ref, yw_ref, dinv_ref, b1_ref, w2_ref, out_ref):
    dinv = dinv_ref[...]
    h = dinv * (acc_ref[0] + acc_ref[1] - yw_ref[...]) + b1_ref[...]
    h = jnp.where(h > 0, h, 0.01 * h)
    hw = jnp.dot(h, w2_ref[...], preferred_element_type=f32)
    out_ref[...] = dinv * hw


_tc2 = pl.pallas_call(
    _tc2_body,
    grid=(GRID,),
    in_specs=[
        pl.BlockSpec((NC, BLK, CH), lambda i: (0, i, 0)),
        pl.BlockSpec((BLK, CH), lambda i: (i, 0)),
        pl.BlockSpec((BLK, 1), lambda i: (i, 0)),
        pl.BlockSpec((1, CH), lambda i: (0, 0)),
        pl.BlockSpec((CH, CH), lambda i: (0, 0)),
    ],
    out_specs=pl.BlockSpec((BLK, CH), lambda i: (i, 0)),
    out_shape=jax.ShapeDtypeStruct((NPAD, CH), f32),
)


def _tc3_body(acc_ref, yw_ref, dinv_ref, b2_ref, batch_ref, w3_ref, b3_ref,
              w4_ref, b4_ref, w5_ref, b5_ref, out_ref, sums_scr, cnts_scr):
    i = pl.program_id(0)
    dinv = dinv_ref[...]
    h = dinv * (acc_ref[0] + acc_ref[1] - yw_ref[...]) + b2_ref[...]
    h = jnp.where(h > 0, h, 0.01 * h)
    gids = lax.broadcasted_iota(jnp.int32, (BLK, N_GRAPHS), 1)
    mask = (batch_ref[...] == gids).astype(f32)
    dn = (((0,), (0,)), ((), ()))
    s_step = lax.dot_general(mask, h, dn, preferred_element_type=f32)
    ones = jnp.ones((BLK, CH), f32)
    c_step = lax.dot_general(mask, ones, dn, preferred_element_type=f32)

    @pl.when(i == 0)
    def _():
        sums_scr[...] = s_step
        cnts_scr[...] = c_step

    @pl.when(i > 0)
    def _():
        sums_scr[...] += s_step
        cnts_scr[...] += c_step

    @pl.when(i == GRID - 1)
    def _():
        g = sums_scr[...] / jnp.maximum(cnts_scr[...], 1.0)
        g = jnp.dot(g, w3_ref[...], preferred_element_type=f32) + b3_ref[...]
        g = jnp.where(g > 0, g, 0.01 * g)
        g = jnp.dot(g, w4_ref[...], preferred_element_type=f32) + b4_ref[...]
        g = jnp.where(g > 0, g, 0.01 * g)
        out_ref[...] = jnp.dot(g, w5_ref[...], preferred_element_type=f32) + b5_ref[...]


_tc3 = pl.pallas_call(
    _tc3_body,
    grid=(GRID,),
    in_specs=[
        pl.BlockSpec((NC, BLK, CH), lambda i: (0, i, 0)),
        pl.BlockSpec((BLK, CH), lambda i: (i, 0)),
        pl.BlockSpec((BLK, 1), lambda i: (i, 0)),
        pl.BlockSpec((1, CH), lambda i: (0, 0)),
        pl.BlockSpec((BLK, 1), lambda i: (i, 0)),
        pl.BlockSpec((CH, CH), lambda i: (0, 0)),
        pl.BlockSpec((1, CH), lambda i: (0, 0)),
        pl.BlockSpec((CH, CH), lambda i: (0, 0)),
        pl.BlockSpec((1, CH), lambda i: (0, 0)),
        pl.BlockSpec((CH, CH), lambda i: (0, 0)),
        pl.BlockSpec((1, CH), lambda i: (0, 0)),
    ],
    out_specs=pl.BlockSpec((N_GRAPHS, CH), lambda i: (0, 0)),
    out_shape=jax.ShapeDtypeStruct((N_GRAPHS, CH), f32),
    scratch_shapes=[
        pltpu.VMEM((N_GRAPHS, CH), f32),
        pltpu.VMEM((N_GRAPHS, CH), f32),
    ],
)


def kernel(x, edge_index, batch, W1, b1, W2, b2, W3, b3, W4, b4, W5, b5):
    pad = EPAD - N_EDGES
    pad_idx = N_NODES + (jnp.arange(pad, dtype=jnp.int32) % 16)
    src_w = jnp.concatenate([edge_index[0], pad_idx]).reshape(NROWS, WIN)
    dst_w = jnp.concatenate([edge_index[1], pad_idx]).reshape(NROWS, WIN)
    eidx = jnp.stack([src_w, dst_w], axis=1)
    del src_w, dst_w
    x_pad = jnp.pad(x, ((0, NPAD - N_NODES), (0, 0)))
    batch_pad = jnp.pad(batch, (0, NPAD - N_NODES),
                        constant_values=N_GRAPHS).reshape(NPAD, 1)
    init0 = (jnp.arange(NPAD) < N_NODES).astype(f32)
    deg_init = jnp.stack([init0, jnp.zeros_like(init0)])
    W3p = jnp.pad(W3, ((0, 0), (0, 64)))
    W4p = jnp.pad(W4, ((0, 64), (0, 64)))
    W5p = jnp.pad(W5, ((0, 64), (0, CH - N_CLASSES)))
    b1r = b1.reshape(1, CH)
    b2r = b2.reshape(1, CH)
    b3p = jnp.pad(b3, (0, 64)).reshape(1, CH)
    b4p = jnp.pad(b4, (0, 64)).reshape(1, CH)
    b5p = jnp.pad(b5, (0, CH - N_CLASSES)).reshape(1, CH)

    _sc_degree, _sc_spmm = _build_sc_kernels()
    xw1 = _tca(x_pad, W1)
    degs = _sc_degree(eidx, deg_init)
    yw1, dinv = _tcb(degs.reshape(NC, NPAD, 1), xw1)
    acc1 = _sc_spmm(eidx, yw1)
    yw2 = _tc2(acc1, yw1, dinv, b1r, W2)
    acc2 = _sc_spmm(eidx, yw2)
    out = _tc3(acc2, yw2, dinv, b2r, batch_pad,
               W3p, b3p, W4p, b4p, W5p, b5p)
    return out[:, :N_CLASSES]


# speedup vs baseline: 1.0084x; 1.0084x over previous
import functools

import jax
import jax.numpy as jnp
from jax import lax
from jax.experimental import pallas as pl
from jax.experimental.pallas import tpu as pltpu
from jax.experimental.pallas import tpu_sc as plsc

N_NODES = 10000
N_EDGES = 320000
CH = 128
N_GRAPHS = 64
N_CLASSES = 10

NC, NS = 2, 16
NW = NC * NS
WIN = 112
NWIN = 90
DEG_GRP = 10
EPAD = NW * NWIN * WIN
NROWS = EPAD // WIN
NPAD = 10240
ROWS_PER_TILE = NPAD // NS

BLK = 1024
GRID = NPAD // BLK

f32 = jnp.float32


@functools.cache
def _build_sc_kernels():
    mesh = plsc.VectorSubcoreMesh(core_axis_name="c", subcore_axis_name="s",
                                  num_cores=NC, num_subcores=NS)

    @functools.partial(
        pl.kernel,
        out_type=jax.ShapeDtypeStruct((NC, NPAD), f32),
        mesh=mesh,
        scratch_types=[
            pltpu.VMEM((NWIN, 2, WIN), jnp.int32),
            pltpu.VMEM((WIN,), f32),
            pltpu.VMEM_SHARED((NPAD,), f32),
            pltpu.SemaphoreType.DMA,
        ],
    )
    def _sc_degree(eidx_hbm, init_hbm, out_hbm, dst_v, ones_v, deg_sh, sem):
        c = lax.axis_index("c")
        s = lax.axis_index("s")
        wid = s * NC + c
        pltpu.sync_copy(eidx_hbm.at[pl.ds(wid * NWIN, NWIN)], dst_v)
        for i in range(WIN // 16):
            ones_v[pl.ds(i * 16, 16)] = jnp.full((16,), 1.0, f32)
        base = s * ROWS_PER_TILE
        pltpu.sync_copy(init_hbm.at[c, pl.ds(base, ROWS_PER_TILE)],
                        deg_sh.at[pl.ds(base, ROWS_PER_TILE)])
        plsc.subcore_barrier()

        def scat(j):
            return pltpu.make_async_copy(ones_v, deg_sh.at[dst_v.at[j, 1]],
                                         sem)

        def body(g, carry):
            for i in range(DEG_GRP):
                scat(g * DEG_GRP + i).start(add=True)
            for i in range(DEG_GRP):
                scat(g * DEG_GRP + i).wait()
            return carry

        lax.fori_loop(0, NWIN // DEG_GRP, body, 0)
        plsc.subcore_barrier()
        pltpu.sync_copy(deg_sh.at[pl.ds(base, ROWS_PER_TILE)],
                        out_hbm.at[c, pl.ds(base, ROWS_PER_TILE)])

    @functools.partial(
        pl.kernel,
        out_type=jax.ShapeDtypeStruct((NC, NPAD, CH), f32),
        mesh=mesh,
        scratch_types=[
            pltpu.VMEM((5, 2, WIN), jnp.int32),
            pltpu.VMEM((3, WIN, CH), f32),
            pltpu.VMEM_SHARED((NPAD, CH), f32),
            pltpu.SemaphoreType.DMA,
            pltpu.SemaphoreType.DMA,
            pltpu.SemaphoreType.DMA,
        ],
    )
    def _sc_spmm(eidx_hbm, yw_hbm, out_hbm, idx_v, rows_v, acc_sh,
                 isem, gsem, ssem):
        c = lax.axis_index("c")
        s = lax.axis_index("s")
        wid = s * NC + c
        row0 = wid * NWIN
        base = s * ROWS_PER_TILE

        def idx_copy(j):
            return pltpu.make_async_copy(eidx_hbm.at[row0 + j],
                                         idx_v.at[j % 5], isem)

        def gather(j):
            return pltpu.make_async_copy(yw_hbm.at[idx_v.at[j % 5, 0]],
                                         rows_v.at[j % 3], gsem)

        def scatter(j):
            return pltpu.make_async_copy(rows_v.at[j % 3],
                                         acc_sh.at[idx_v.at[j % 5, 1]], ssem)

        idx_copy(0).start()
        idx_copy(1).start()
        pltpu.sync_copy(yw_hbm.at[pl.ds(base, ROWS_PER_TILE)],
                        acc_sh.at[pl.ds(base, ROWS_PER_TILE)])
        plsc.subcore_barrier()
        idx_copy(0).wait()
        gather(0).start()

        def body(j, carry):
            @pl.when(j >= 2)
            def _():
                scatter(j - 2).wait()

            @pl.when(j + 2 < NWIN)
            def _():
                idx_copy(j + 2).start()

            @pl.when(j + 1 < NWIN)
            def _():
                idx_copy(j + 1).wait()
                gather(j + 1).start()

            gather(j).wait()
            scatter(j).start(add=True)
            return carry

        lax.fori_loop(0, NWIN, body, 0)

        @pl.when(NWIN >= 2)
        def _():
            scatter(NWIN - 2).wait()

        scatter(NWIN - 1).wait()
        plsc.subcore_barrier()
        pltpu.sync_copy(acc_sh.at[pl.ds(base, ROWS_PER_TILE)],
                        out_hbm.at[c, pl.ds(base, ROWS_PER_TILE)])

    return _sc_degree, _sc_spmm


def _tc1_body(deg_ref, x_ref, w1_ref, yw_ref, dinv_ref):
    deg = deg_ref[0] + deg_ref[1]
    dinv = jnp.where(deg > 0, lax.rsqrt(deg), 0.0)
    dinv_ref[...] = dinv
    xw = jnp.dot(x_ref[...], w1_ref[...], preferred_element_type=f32)
    yw_ref[...] = dinv * xw


_tc1 = pl.pallas_call(
    _tc1_body,
    grid=(GRID,),
    in_specs=[
        pl.BlockSpec((NC, BLK, 1), lambda i: (0, i, 0)),
        pl.BlockSpec((BLK, CH), lambda i: (i, 0)),
        pl.BlockSpec((CH, CH), lambda i: (0, 0)),
    ],
    out_specs=[
        pl.BlockSpec((BLK, CH), lambda i: (i, 0)),
        pl.BlockSpec((BLK, 1), lambda i: (i, 0)),
    ],
    out_shape=[
        jax.ShapeDtypeStruct((NPAD, CH), f32),
        jax.ShapeDtypeStruct((NPAD, 1), f32),
    ],
)


def _tc2_body(acc_ref, yw_ref, dinv_ref, b1_ref, w2_ref, out_ref):
    dinv = dinv_ref[...]
    h = dinv * (acc_ref[0] + acc_ref[1] - yw_ref[...]) + b1_ref[...]
    h = jnp.where(h > 0, h, 0.01 * h)
    hw = jnp.dot(h, w2_ref[...], preferred_element_type=f32)
    out_ref[...] = dinv * hw


_tc2 = pl.pallas_call(
    _tc2_body,
    grid=(GRID,),
    in_specs=[
        pl.BlockSpec((NC, BLK, CH), lambda i: (0, i, 0)),
        pl.BlockSpec((BLK, CH), lambda i: (i, 0)),
        pl.BlockSpec((BLK, 1), lambda i: (i, 0)),
        pl.BlockSpec((1, CH), lambda i: (0, 0)),
        pl.BlockSpec((CH, CH), lambda i: (0, 0)),
    ],
    out_specs=pl.BlockSpec((BLK, CH), lambda i: (i, 0)),
    out_shape=jax.ShapeDtypeStruct((NPAD, CH), f32),
)


def _tc3_body(acc_ref, yw_ref, dinv_ref, b2_ref, batch_ref, w3_ref, b3_ref,
              w4_ref, b4_ref, w5_ref, b5_ref, out_ref, sums_scr, cnts_scr):
    i = pl.program_id(0)
    dinv = dinv_ref[...]
    h = dinv * (acc_ref[0] + acc_ref[1] - yw_ref[...]) + b2_ref[...]
    h = jnp.where(h > 0, h, 0.01 * h)
    gids = lax.broadcasted_iota(jnp.int32, (BLK, N_GRAPHS), 1)
    mask = (batch_ref[...] == gids).astype(f32)
    dn = (((0,), (0,)), ((), ()))
    s_step = lax.dot_general(mask, h, dn, preferred_element_type=f32)
    ones = jnp.ones((BLK, CH), f32)
    c_step = lax.dot_general(mask, ones, dn, preferred_element_type=f32)

    @pl.when(i == 0)
    def _():
        sums_scr[...] = s_step
        cnts_scr[...] = c_step

    @pl.when(i > 0)
    def _():
        sums_scr[...] += s_step
        cnts_scr[...] += c_step

    @pl.when(i == GRID - 1)
    def _():
        g = sums_scr[...] / jnp.maximum(cnts_scr[...], 1.0)
        g = jnp.dot(g, w3_ref[...], preferred_element_type=f32) + b3_ref[...]
        g = jnp.where(g > 0, g, 0.01 * g)
        g = jnp.dot(g, w4_ref[...], preferred_element_type=f32) + b4_ref[...]
        g = jnp.where(g > 0, g, 0.01 * g)
        out_ref[...] = jnp.dot(g, w5_ref[...], preferred_element_type=f32) + b5_ref[...]


_tc3 = pl.pallas_call(
    _tc3_body,
    grid=(GRID,),
    in_specs=[
        pl.BlockSpec((NC, BLK, CH), lambda i: (0, i, 0)),
        pl.BlockSpec((BLK, CH), lambda i: (i, 0)),
        pl.BlockSpec((BLK, 1), lambda i: (i, 0)),
        pl.BlockSpec((1, CH), lambda i: (0, 0)),
        pl.BlockSpec((BLK, 1), lambda i: (i, 0)),
        pl.BlockSpec((CH, CH), lambda i: (0, 0)),
        pl.BlockSpec((1, CH), lambda i: (0, 0)),
        pl.BlockSpec((CH, CH), lambda i: (0, 0)),
        pl.BlockSpec((1, CH), lambda i: (0, 0)),
        pl.BlockSpec((CH, CH), lambda i: (0, 0)),
        pl.BlockSpec((1, CH), lambda i: (0, 0)),
    ],
    out_specs=pl.BlockSpec((N_GRAPHS, CH), lambda i: (0, 0)),
    out_shape=jax.ShapeDtypeStruct((N_GRAPHS, CH), f32),
    scratch_shapes=[
        pltpu.VMEM((N_GRAPHS, CH), f32),
        pltpu.VMEM((N_GRAPHS, CH), f32),
    ],
)


def kernel(x, edge_index, batch, W1, b1, W2, b2, W3, b3, W4, b4, W5, b5):
    pad = EPAD - N_EDGES
    pad_idx = N_NODES + (jnp.arange(pad, dtype=jnp.int32) % 16)
    src_w = jnp.concatenate([edge_index[0], pad_idx]).reshape(NROWS, WIN)
    dst_w = jnp.concatenate([edge_index[1], pad_idx]).reshape(NROWS, WIN)
    eidx = jnp.stack([src_w, dst_w], axis=1)
    del src_w, dst_w
    x_pad = jnp.pad(x, ((0, NPAD - N_NODES), (0, 0)))
    batch_pad = jnp.pad(batch, (0, NPAD - N_NODES),
                        constant_values=N_GRAPHS).reshape(NPAD, 1)
    init0 = (jnp.arange(NPAD) < N_NODES).astype(f32)
    deg_init = jnp.stack([init0, jnp.zeros_like(init0)])
    W3p = jnp.pad(W3, ((0, 0), (0, 64)))
    W4p = jnp.pad(W4, ((0, 64), (0, 64)))
    W5p = jnp.pad(W5, ((0, 64), (0, CH - N_CLASSES)))
    b1r = b1.reshape(1, CH)
    b2r = b2.reshape(1, CH)
    b3p = jnp.pad(b3, (0, 64)).reshape(1, CH)
    b4p = jnp.pad(b4, (0, 64)).reshape(1, CH)
    b5p = jnp.pad(b5, (0, CH - N_CLASSES)).reshape(1, CH)

    _sc_degree, _sc_spmm = _build_sc_kernels()
    degs = _sc_degree(eidx, deg_init)
    yw1, dinv = _tc1(degs.reshape(NC, NPAD, 1), x_pad, W1)
    acc1 = _sc_spmm(eidx, yw1)
    yw2 = _tc2(acc1, yw1, dinv, b1r, W2)
    acc2 = _sc_spmm(eidx, yw2)
    out = _tc3(acc2, yw2, dinv, b2r, batch_pad,
               W3p, b3p, W4p, b4p, W5p, b5p)
    return out[:, :N_CLASSES]
